# Initial kernel scaffold; baseline (speedup 1.0000x reference)
import jax
import jax.numpy as jnp
from jax.experimental import pallas as pl


def kernel(x_input, train_inputs, train_labels, y_labels, features):
    raise NotImplementedError("write your pallas kernel here")


# speedup vs baseline: 2.5683x; 2.5683x over previous
import functools

import jax
import jax.numpy as jnp
from jax import lax
from jax.experimental import pallas as pl
from jax.experimental.pallas import tpu as pltpu

_B = 1024
_N = 100000
_D = 16
_K = 3
_NUM_LABELS = 10
_PRED_TOL = 0.25

_NT = 2048
_T = (_N + _NT - 1) // _NT
_NPAD = _T * _NT

_NEG_INF = float("-inf")
_BIG_I = 2**30


def _topk_body(x_ref, t_ref, f_ref, vals_ref, idx_ref, run_v, run_i):
    nt = pl.program_id(0)

    @pl.when(nt == 0)
    def _init():
        run_v[...] = jnp.full((_B, 8), _NEG_INF, jnp.float32)
        run_i[...] = jnp.full((_B, 8), _BIG_I, jnp.int32)

    x = x_ref[...]
    t = t_ref[...]
    f = f_ref[...]
    tw = t * f
    st = jnp.sum(tw * tw, axis=1)
    sx = jnp.sum(x * x, axis=1)
    cross = lax.dot_general(
        x, tw, (((1,), (1,)), ((), ())),
        preferred_element_type=jnp.float32)
    neg = 2.0 * cross - jnp.sqrt(sx[:, None] + st[None, :])
    gidx = nt * _NT + lax.broadcasted_iota(jnp.int32, (_B, _NT), 1)
    neg = jnp.where(gidx < _N, neg, _NEG_INF)

    tv, ti = [], []
    for k in range(_K):
        m = jnp.max(neg, axis=1, keepdims=True)
        p = jnp.min(jnp.where(neg == m, gidx, _BIG_I), axis=1,
                    keepdims=True)
        tv.append(m)
        ti.append(p)
        if k < _K - 1:
            neg = jnp.where(gidx == p, _NEG_INF, neg)

    lane8 = lax.broadcasted_iota(jnp.int32, (_B, 8), 1)
    cv = run_v[...]
    ci = run_i[...]
    for k in range(_K):
        cv = jnp.where(lane8 == 3 + k, tv[k], cv)
        ci = jnp.where(lane8 == 3 + k, ti[k], ci)
    bv, bi = [], []
    for k in range(_K):
        m = jnp.max(cv, axis=1, keepdims=True)
        p = jnp.min(jnp.where(cv == m, ci, _BIG_I), axis=1, keepdims=True)
        bv.append(m)
        bi.append(p)
        cv = jnp.where(ci == p, _NEG_INF, cv)
    nv = jnp.where(lane8 == 0, bv[0],
                   jnp.where(lane8 == 1, bv[1],
                             jnp.where(lane8 == 2, bv[2], _NEG_INF)))
    ni = jnp.where(lane8 == 0, bi[0],
                   jnp.where(lane8 == 1, bi[1],
                             jnp.where(lane8 == 2, bi[2], _BIG_I)))
    run_v[...] = nv
    run_i[...] = ni

    @pl.when(nt == _T - 1)
    def _fin():
        i3 = lax.broadcasted_iota(jnp.int32, (_B, _K), 1)
        vals_ref[...] = jnp.where(i3 == 0, bv[0],
                                  jnp.where(i3 == 1, bv[1], bv[2]))
        idx_ref[...] = jnp.where(i3 == 0, bi[0],
                                 jnp.where(i3 == 1, bi[1], bi[2]))


@functools.partial(jax.jit, static_argnames=("interpret",))
def _tc_topk(x, train, features, interpret=False):
    t_pad = jnp.pad(train, ((0, _NPAD - _N), (0, 0)))
    f2 = features.reshape(1, _D)
    return pl.pallas_call(
        _topk_body,
        grid=(_T,),
        in_specs=[
            pl.BlockSpec((_B, _D), lambda n: (0, 0)),
            pl.BlockSpec((_NT, _D), lambda n: (n, 0)),
            pl.BlockSpec((1, _D), lambda n: (0, 0)),
        ],
        out_specs=[
            pl.BlockSpec((_B, _K), lambda n: (0, 0)),
            pl.BlockSpec((_B, _K), lambda n: (0, 0)),
        ],
        out_shape=[
            jax.ShapeDtypeStruct((_B, _K), jnp.float32),
            jax.ShapeDtypeStruct((_B, _K), jnp.int32),
        ],
        scratch_shapes=[
            pltpu.VMEM((_B, 8), jnp.float32),
            pltpu.VMEM((_B, 8), jnp.int32),
        ],
        compiler_params=pltpu.CompilerParams(
            dimension_semantics=("arbitrary",)),
        interpret=interpret,
    )(x, t_pad, f2)


def kernel(x_input, train_inputs, train_labels, y_labels, features):
    values, indices = _tc_topk(x_input, train_inputs, features)
    labels = jnp.take(train_labels, indices, axis=0)
    outputs = jnp.sum(labels, axis=1) // _K
    one_hot_out = jax.nn.one_hot(outputs, _NUM_LABELS, dtype=jnp.float32)
    magnitude = (jnp.abs((y_labels - outputs).astype(jnp.float32))
                 / (y_labels + 1).astype(jnp.float32))
    pred = (jnp.sum((magnitude < _PRED_TOL).astype(jnp.int32)).astype(jnp.float32)
            / jnp.float32(_B))
    return values, indices, labels, one_hot_out, pred
